# baseline (speedup 1.0000x reference)
import functools

import jax
import jax.numpy as jnp
from jax import lax
from jax.experimental import pallas as pl
from jax.experimental.pallas import tpu as pltpu
from jax.experimental.pallas import tpu_sc as plsc

N = 10000
E = 320000
IN_DIM = 128
HID = 16
HEADS = 8
D1 = HEADS * HID
D2 = 16

NC = 2
NS = 16
NW = NC * NS
CH = 128
NCHUNK = E // CH
CH1 = 100
NCHUNK1 = E // CH1
CPT1 = NCHUNK1 // NS
ROWS_A = 624
ROWS_LAST = N - (NS - 1) * ROWS_A


def _tile_copy(s, copy_fn):
    r0 = s * ROWS_A

    @pl.when(s < NS - 1)
    def _():
        copy_fn(r0, ROWS_A)

    @pl.when(s == NS - 1)
    def _():
        copy_fn(r0, ROWS_LAST)

_Q, _R = divmod(NCHUNK, NW)

f32 = jnp.float32
i32 = jnp.int32


_BLK = 1000
_GRID = N // _BLK


def _tc1_body(x_ref, w1_ref, acat_ref, h1a_ref, h1b_ref, a1_ref, a1r_ref,
              bmax_ref):
    i = pl.program_id(0)
    h = jnp.dot(x_ref[...], w1_ref[...], preferred_element_type=f32)
    h1a_ref[...] = h[:, 0:64]
    h1b_ref[...] = h[:, 64:128]
    a1 = jnp.dot(h, acat_ref[...], preferred_element_type=f32)
    a1_ref[...] = a1
    a1r_ref[...] = jnp.concatenate([a1[:, 8:], a1[:, :8]], axis=1)
    m = jnp.broadcast_to(jnp.max(a1, axis=0, keepdims=True), (8, 16))

    @pl.when(i == 0)
    def _():
        bmax_ref[...] = m

    @pl.when(i != 0)
    def _():
        bmax_ref[...] = jnp.maximum(bmax_ref[...], m)


def _tc1(x, w1, acat):
    return pl.pallas_call(
        _tc1_body,
        grid=(_GRID,),
        in_specs=[
            pl.BlockSpec((_BLK, IN_DIM), lambda i: (i, 0)),
            pl.BlockSpec((IN_DIM, D1), lambda i: (0, 0)),
            pl.BlockSpec((D1, 16), lambda i: (0, 0)),
        ],
        out_specs=[
            pl.BlockSpec((_BLK, 64), lambda i: (i, 0)),
            pl.BlockSpec((_BLK, 64), lambda i: (i, 0)),
            pl.BlockSpec((_BLK, 16), lambda i: (i, 0)),
            pl.BlockSpec((_BLK, 16), lambda i: (i, 0)),
            pl.BlockSpec((8, 16), lambda i: (0, 0)),
        ],
        out_shape=[
            jax.ShapeDtypeStruct((N, 64), f32),
            jax.ShapeDtypeStruct((N, 64), f32),
            jax.ShapeDtypeStruct((N, 16), f32),
            jax.ShapeDtypeStruct((N, 16), f32),
            jax.ShapeDtypeStruct((8, 16), f32),
        ],
    )(x, w1, acat)


def _tc2_body(acc0_ref, acc1_ref, e8_ref, b1_ref, w2_ref, a2m_ref,
              h2_ref, a2_ref, bmax2_ref):
    i = pl.program_id(0)
    a0 = acc0_ref[0]
    a1_ = acc1_ref[0]
    num = jnp.concatenate([a0[:, 0:64], a1_[:, 0:64]], axis=1)
    den8 = a0[:, 64:72]
    den = jnp.dot(den8, e8_ref[...], preferred_element_type=f32)
    out1 = num / (den + 1e-16)
    z = out1 + b1_ref[...]
    z = jnp.where(z > 0, z, jnp.exp(jnp.minimum(z, 0.0)) - 1.0)
    h2 = jnp.dot(z, w2_ref[...], preferred_element_type=f32)
    h2_ref[...] = h2
    a2 = jnp.dot(h2, a2m_ref[...], preferred_element_type=f32)
    a2_ref[...] = a2
    m = jnp.broadcast_to(jnp.max(a2, axis=0, keepdims=True), (8, 16))

    @pl.when(i == 0)
    def _():
        bmax2_ref[...] = m

    @pl.when(i != 0)
    def _():
        bmax2_ref[...] = jnp.maximum(bmax2_ref[...], m)


def _tc2(acc, e8, b1, w2, a2m):
    return pl.pallas_call(
        _tc2_body,
        grid=(_GRID,),
        in_specs=[
            pl.BlockSpec((1, _BLK, 80), lambda i: (0, i, 0)),
            pl.BlockSpec((1, _BLK, 80), lambda i: (1, i, 0)),
            pl.BlockSpec((8, D1), lambda i: (0, 0)),
            pl.BlockSpec((1, D1), lambda i: (0, 0)),
            pl.BlockSpec((D1, D2), lambda i: (0, 0)),
            pl.BlockSpec((D2, 16), lambda i: (0, 0)),
        ],
        out_specs=[
            pl.BlockSpec((_BLK, D2), lambda i: (i, 0)),
            pl.BlockSpec((_BLK, 16), lambda i: (i, 0)),
            pl.BlockSpec((8, 16), lambda i: (0, 0)),
        ],
        out_shape=[
            jax.ShapeDtypeStruct((N, D2), f32),
            jax.ShapeDtypeStruct((N, 16), f32),
            jax.ShapeDtypeStruct((8, 16), f32),
        ],
    )(acc, acc, e8, b1, w2, a2m)


def _tc3_body(acc0_ref, acc1_ref, b2_ref, out_ref):
    a = acc0_ref[0] + acc1_ref[0]
    out_ref[...] = a[:, 0:16] / (a[:, 16:32] + 1e-16) + b2_ref[...]


def _tc3(acc2, b2):
    return pl.pallas_call(
        _tc3_body,
        grid=(_GRID,),
        in_specs=[
            pl.BlockSpec((1, _BLK, 32), lambda i: (0, i, 0)),
            pl.BlockSpec((1, _BLK, 32), lambda i: (1, i, 0)),
            pl.BlockSpec((1, D2), lambda i: (0, 0)),
        ],
        out_specs=pl.BlockSpec((_BLK, D2), lambda i: (i, 0)),
        out_shape=jax.ShapeDtypeStruct((N, D2), f32),
    )(acc2, acc2, b2)


_PROBE_NO_SCATTER1 = False
_PROBE_NO_COMPUTE1 = True


def _sc_edge_pass1(src2, dst2, a1, a1r, h1a, h1b, btile, z80):
    mesh = plsc.VectorSubcoreMesh(
        core_axis_name="c", subcore_axis_name="s", num_cores=NC,
        num_subcores=NS)

    scratch = [
        pltpu.VMEM((CPT1, CH1), i32),
        pltpu.VMEM((CPT1, CH1), i32),
        pltpu.VMEM((16,), f32),
        pltpu.VMEM_SHARED((N, 80), f32),
    ]
    for _ in range(2):
        scratch += [
            pltpu.VMEM((CH1, 16), f32),
            pltpu.VMEM((CH1, 16), f32),
            pltpu.VMEM((CH1, 64), f32),
            pltpu.VMEM((CH1, 80), f32),
            pltpu.SemaphoreType.DMA,
            pltpu.SemaphoreType.DMA,
            pltpu.SemaphoreType.DMA,
        ]

    @functools.partial(
        pl.kernel,
        out_type=jax.ShapeDtypeStruct((NC, N, 80), f32),
        mesh=mesh,
        scratch_types=scratch,
        compiler_params=pltpu.CompilerParams(
            use_tc_tiling_on_sc=False, needs_layout_passes=False),
    )
    def k(src2_hbm, dst2_hbm, a1_hbm, a1r_hbm, h1a_hbm, h1b_hbm, btile_hbm,
          z80_hbm,
          acc_out, sidx_all, didx_all, bvec, sacc,
          arow0, brow0, hrow0, msg0, s00, s01, s02,
          arow1, brow1, hrow1, msg1, s10, s11, s12):
        c = lax.axis_index("c")
        s = lax.axis_index("s")
        arows = (arow0, arow1)
        brows = (brow0, brow1)
        hrows = (hrow0, hrow1)
        msgs = (msg0, msg1)
        sems = ((s00, s01, s02), (s10, s11, s12))
        pltpu.sync_copy(btile_hbm, bvec)
        base = s * CPT1
        pltpu.sync_copy(src2_hbm.at[pl.ds(base, CPT1)], sidx_all)
        pltpu.sync_copy(dst2_hbm.at[pl.ds(base, CPT1)], didx_all)
        _tile_copy(s, lambda r0, nr: pltpu.sync_copy(
            z80_hbm.at[pl.ds(r0, nr)], sacc.at[pl.ds(r0, nr)]))
        plsc.subcore_barrier()
        bv = bvec[...]

        def run_all(cc, h1h_hbm):
            def issue(kk, ab):
                return (
                    pltpu.async_copy(a1_hbm.at[sidx_all.at[kk]], arows[ab],
                                     sems[ab][0]),
                    pltpu.async_copy(a1r_hbm.at[didx_all.at[kk]], brows[ab],
                                     sems[ab][1]),
                    pltpu.async_copy(h1h_hbm.at[sidx_all.at[kk]], hrows[ab],
                                     sems[ab][2]),
                )

            def run_chunk(kk, ab, cps):
                for cp in cps:
                    cp.wait()
                arow, brow = arows[ab], brows[ab]
                hrow, msg = hrows[ab], msgs[ab]

                if not _PROBE_NO_COMPUTE1:
                    @plsc.parallel_loop(0, CH1, unroll=4)
                    def _(e):
                        al = arow[e, :] + brow[e, :]
                        al = jnp.maximum(al, 0.2 * al)
                        ex = jnp.exp(al - bv)
                        msg[e, pl.ds(64, 16)] = ex
                        for hh in range(4):
                            sp = jnp.full((16,), ex[4 * cc + hh], f32)
                            msg[e, pl.ds(hh * 16, 16)] = (
                                hrow[e, pl.ds(hh * 16, 16)] * sp)

                if not _PROBE_NO_SCATTER1:
                    pltpu.sync_copy(msg, sacc.at[didx_all.at[kk]], add=True)

            def pair_body(p, carry):
                ka = 2 * p
                kb = 2 * p + 1
                cpa = issue(ka, 0)
                cpb = issue(kb, 1)
                run_chunk(ka, 0, cpa)
                run_chunk(kb, 1, cpb)
                return carry

            lax.fori_loop(0, CPT1 // 2, pair_body, 0)

        @pl.when(c == 0)
        def _():
            run_all(0, h1a_hbm)

        @pl.when(c == 1)
        def _():
            run_all(1, h1b_hbm)

        plsc.subcore_barrier()
        _tile_copy(s, lambda r0, nr: pltpu.sync_copy(
            sacc.at[pl.ds(r0, nr)], acc_out.at[c, pl.ds(r0, nr)]))

    return k(src2, dst2, a1, a1r, h1a, h1b, btile, z80)


def _sc_edge_pass2(src2, dst2, as2, ad2, h2, b2t, z32):
    mesh = plsc.VectorSubcoreMesh(
        core_axis_name="c", subcore_axis_name="s", num_cores=NC,
        num_subcores=NS)

    scratch = [
        pltpu.VMEM((_Q + 1, CH), i32),
        pltpu.VMEM((_Q + 1, CH), i32),
        pltpu.VMEM((N,), f32),
        pltpu.VMEM((N,), f32),
        pltpu.VMEM((16,), f32),
        pltpu.VMEM_SHARED((N, 32), f32),
    ]
    for _ in range(2):
        scratch += [
            pltpu.VMEM((CH, D2), f32),
            pltpu.VMEM((CH, 32), f32),
            pltpu.VMEM((CH,), f32),
            pltpu.SemaphoreType.DMA,
        ]

    @functools.partial(
        pl.kernel,
        out_type=jax.ShapeDtypeStruct((NC, N, 32), f32),
        mesh=mesh,
        scratch_types=scratch,
        compiler_params=pltpu.CompilerParams(
            use_tc_tiling_on_sc=False, needs_layout_passes=False),
    )
    def k(src2_hbm, dst2_hbm, as2_hbm, ad2_hbm, h2_hbm, b2t_hbm, z32_hbm,
          acc_out, sidx_all, didx_all, ast, adt, b2v, sacc2,
          hrow20, msg20, exbuf0, sm0,
          hrow21, msg21, exbuf1, sm1):
        c = lax.axis_index("c")
        s = lax.axis_index("s")
        wid = s * NC + c
        hrows = (hrow20, hrow21)
        msgs = (msg20, msg21)
        exbufs = (exbuf0, exbuf1)
        sems = (sm0, sm1)
        pltpu.sync_copy(b2t_hbm, b2v)
        pltpu.sync_copy(as2_hbm, ast)
        pltpu.sync_copy(ad2_hbm, adt)
        base = wid * _Q
        pltpu.sync_copy(src2_hbm.at[pl.ds(base, _Q)],
                        sidx_all.at[pl.ds(0, _Q)])
        pltpu.sync_copy(dst2_hbm.at[pl.ds(base, _Q)],
                        didx_all.at[pl.ds(0, _Q)])

        @pl.when(wid < _R)
        def _():
            pltpu.sync_copy(src2_hbm.at[NW * _Q + wid], sidx_all.at[_Q])
            pltpu.sync_copy(dst2_hbm.at[NW * _Q + wid], didx_all.at[_Q])

        _tile_copy(s, lambda r0, nr: pltpu.sync_copy(
            z32_hbm.at[pl.ds(r0, nr)], sacc2.at[pl.ds(r0, nr)]))
        plsc.subcore_barrier()
        bv = b2v[...]

        def issue(kk, ab):
            return pltpu.async_copy(h2_hbm.at[sidx_all.at[kk]], hrows[ab],
                                    sems[ab])

        def run_chunk(kk, ab, cp):
            hrow2, msg2, exbuf = hrows[ab], msgs[ab], exbufs[ab]

            @plsc.parallel_loop(0, CH // 16, unroll=2)
            def _(gi):
                sv = sidx_all[kk, pl.ds(gi * 16, 16)]
                dv = didx_all[kk, pl.ds(gi * 16, 16)]
                al = plsc.load_gather(ast, [sv]) + plsc.load_gather(adt, [dv])
                al = jnp.maximum(al, 0.2 * al)
                exbuf[pl.ds(gi * 16, 16)] = jnp.exp(al - bv)

            cp.wait()

            @plsc.parallel_loop(0, CH // 16, unroll=2)
            def _(gi):
                exv = exbuf[pl.ds(gi * 16, 16)]
                for j in range(16):
                    e = gi * 16 + j
                    sp = jnp.full((16,), exv[j], f32)
                    msg2[e, pl.ds(0, 16)] = hrow2[e, :] * sp
                    msg2[e, pl.ds(16, 16)] = sp

            pltpu.sync_copy(msg2, sacc2.at[didx_all.at[kk]], add=True)

        def pair_body(p, carry):
            ka = 2 * p
            kb = 2 * p + 1
            cpa = issue(ka, 0)
            cpb = issue(kb, 1)
            run_chunk(ka, 0, cpa)
            run_chunk(kb, 1, cpb)
            return carry

        lax.fori_loop(0, _Q // 2, pair_body, 0)

        @pl.when(wid < _R)
        def _():
            run_chunk(_Q, 0, issue(_Q, 0))

        plsc.subcore_barrier()
        _tile_copy(s, lambda r0, nr: pltpu.sync_copy(
            sacc2.at[pl.ds(r0, nr)], acc_out.at[c, pl.ds(r0, nr)]))

    return k(src2, dst2, as2, ad2, h2, b2t, z32)


def kernel(x, adj, W1, att_src1, att_dst1, b1, W2, att_src2, att_dst2, b2):
    src = adj[0].astype(i32)
    dst = adj[1].astype(i32)
    src2a = src.reshape(NCHUNK1, CH1)
    dst2a = dst.reshape(NCHUNK1, CH1)
    src2 = src.reshape(NCHUNK, CH)
    dst2 = dst.reshape(NCHUNK, CH)

    eye8 = jnp.eye(HEADS, dtype=f32)
    m_src = (att_src1[0][:, :, None] * eye8[:, None, :]).reshape(D1, HEADS)
    m_dst = (att_dst1[0][:, :, None] * eye8[:, None, :]).reshape(D1, HEADS)
    acat = jnp.concatenate([m_src, m_dst], axis=1)

    h1a, h1b, a1, a1r, bmax1 = _tc1(x, W1, acat)

    bsum = bmax1[0, :8] + bmax1[0, 8:]
    bh = jnp.maximum(bsum, 0.2 * bsum)
    btile = jnp.tile(bh, 2)

    z80 = jnp.zeros((N, 80), f32)
    acc = _sc_edge_pass1(src2a, dst2a, a1, a1r, h1a, h1b, btile, z80)

    e8 = jnp.kron(eye8, jnp.ones((1, HID), f32))
    a2m = jnp.concatenate(
        [att_src2[0, 0][:, None], att_dst2[0, 0][:, None],
         jnp.zeros((D2, 14), f32)], axis=1)
    h2, a2, bmax2 = _tc2(acc, e8, b1.reshape(1, D1), W2, a2m)

    b2sum = bmax2[0, 0] + bmax2[0, 1]
    b2b = jnp.maximum(b2sum, 0.2 * b2sum)
    b2t = jnp.full((16,), b2b, f32)
    as2 = a2[:, 0] + 0.0
    ad2 = a2[:, 1] + 0.0

    z32 = jnp.zeros((N, 32), f32)
    acc2 = _sc_edge_pass2(src2, dst2, as2, ad2, h2, b2t, z32)

    return _tc3(acc2, b2.reshape(1, D2))


# speedup vs baseline: 1.0477x; 1.0477x over previous
import functools

import jax
import jax.numpy as jnp
from jax import lax
from jax.experimental import pallas as pl
from jax.experimental.pallas import tpu as pltpu
from jax.experimental.pallas import tpu_sc as plsc

N = 10000
E = 320000
IN_DIM = 128
HID = 16
HEADS = 8
D1 = HEADS * HID
D2 = 16

NC = 2
NS = 16
NW = NC * NS
CH = 128
NCHUNK = E // CH
CH1 = 100
NCHUNK1 = E // CH1
CPT1 = NCHUNK1 // NS
ROWS_A = 624
ROWS_LAST = N - (NS - 1) * ROWS_A


def _tile_copy(s, copy_fn):
    r0 = s * ROWS_A

    @pl.when(s < NS - 1)
    def _():
        copy_fn(r0, ROWS_A)

    @pl.when(s == NS - 1)
    def _():
        copy_fn(r0, ROWS_LAST)

_Q, _R = divmod(NCHUNK, NW)

f32 = jnp.float32
i32 = jnp.int32


_BLK = 1000
_GRID = N // _BLK


def _tc1_body(x_ref, w1_ref, acat_ref, h1a_ref, h1b_ref, a1_ref, a1r_ref,
              bmax_ref):
    i = pl.program_id(0)
    h = jnp.dot(x_ref[...], w1_ref[...], preferred_element_type=f32)
    h1a_ref[...] = h[:, 0:64]
    h1b_ref[...] = h[:, 64:128]
    a1 = jnp.dot(h, acat_ref[...], preferred_element_type=f32)
    a1_ref[...] = a1
    a1r_ref[...] = jnp.concatenate([a1[:, 8:], a1[:, :8]], axis=1)
    m = jnp.broadcast_to(jnp.max(a1, axis=0, keepdims=True), (8, 16))

    @pl.when(i == 0)
    def _():
        bmax_ref[...] = m

    @pl.when(i != 0)
    def _():
        bmax_ref[...] = jnp.maximum(bmax_ref[...], m)


def _tc1(x, w1, acat):
    return pl.pallas_call(
        _tc1_body,
        grid=(_GRID,),
        in_specs=[
            pl.BlockSpec((_BLK, IN_DIM), lambda i: (i, 0)),
            pl.BlockSpec((IN_DIM, D1), lambda i: (0, 0)),
            pl.BlockSpec((D1, 16), lambda i: (0, 0)),
        ],
        out_specs=[
            pl.BlockSpec((_BLK, 64), lambda i: (i, 0)),
            pl.BlockSpec((_BLK, 64), lambda i: (i, 0)),
            pl.BlockSpec((_BLK, 16), lambda i: (i, 0)),
            pl.BlockSpec((_BLK, 16), lambda i: (i, 0)),
            pl.BlockSpec((8, 16), lambda i: (0, 0)),
        ],
        out_shape=[
            jax.ShapeDtypeStruct((N, 64), f32),
            jax.ShapeDtypeStruct((N, 64), f32),
            jax.ShapeDtypeStruct((N, 16), f32),
            jax.ShapeDtypeStruct((N, 16), f32),
            jax.ShapeDtypeStruct((8, 16), f32),
        ],
    )(x, w1, acat)


def _tc2_body(acc0_ref, acc1_ref, e8_ref, b1_ref, w2_ref, a2m_ref,
              h2_ref, a2_ref, bmax2_ref):
    i = pl.program_id(0)
    a0 = acc0_ref[0]
    a1_ = acc1_ref[0]
    num = jnp.concatenate([a0[:, 0:64], a1_[:, 0:64]], axis=1)
    den8 = a0[:, 64:72]
    den = jnp.dot(den8, e8_ref[...], preferred_element_type=f32)
    out1 = num / (den + 1e-16)
    z = out1 + b1_ref[...]
    z = jnp.where(z > 0, z, jnp.exp(jnp.minimum(z, 0.0)) - 1.0)
    h2 = jnp.dot(z, w2_ref[...], preferred_element_type=f32)
    h2_ref[...] = h2
    a2 = jnp.dot(h2, a2m_ref[...], preferred_element_type=f32)
    a2_ref[...] = a2
    m = jnp.broadcast_to(jnp.max(a2, axis=0, keepdims=True), (8, 16))

    @pl.when(i == 0)
    def _():
        bmax2_ref[...] = m

    @pl.when(i != 0)
    def _():
        bmax2_ref[...] = jnp.maximum(bmax2_ref[...], m)


def _tc2(acc, e8, b1, w2, a2m):
    return pl.pallas_call(
        _tc2_body,
        grid=(_GRID,),
        in_specs=[
            pl.BlockSpec((1, _BLK, 80), lambda i: (0, i, 0)),
            pl.BlockSpec((1, _BLK, 80), lambda i: (1, i, 0)),
            pl.BlockSpec((8, D1), lambda i: (0, 0)),
            pl.BlockSpec((1, D1), lambda i: (0, 0)),
            pl.BlockSpec((D1, D2), lambda i: (0, 0)),
            pl.BlockSpec((D2, 16), lambda i: (0, 0)),
        ],
        out_specs=[
            pl.BlockSpec((_BLK, D2), lambda i: (i, 0)),
            pl.BlockSpec((_BLK, 16), lambda i: (i, 0)),
            pl.BlockSpec((8, 16), lambda i: (0, 0)),
        ],
        out_shape=[
            jax.ShapeDtypeStruct((N, D2), f32),
            jax.ShapeDtypeStruct((N, 16), f32),
            jax.ShapeDtypeStruct((8, 16), f32),
        ],
    )(acc, acc, e8, b1, w2, a2m)


def _tc3_body(acc0_ref, acc1_ref, b2_ref, out_ref):
    a = acc0_ref[0] + acc1_ref[0]
    out_ref[...] = a[:, 0:16] / (a[:, 16:32] + 1e-16) + b2_ref[...]


def _tc3(acc2, b2):
    return pl.pallas_call(
        _tc3_body,
        grid=(_GRID,),
        in_specs=[
            pl.BlockSpec((1, _BLK, 32), lambda i: (0, i, 0)),
            pl.BlockSpec((1, _BLK, 32), lambda i: (1, i, 0)),
            pl.BlockSpec((1, D2), lambda i: (0, 0)),
        ],
        out_specs=pl.BlockSpec((_BLK, D2), lambda i: (i, 0)),
        out_shape=jax.ShapeDtypeStruct((N, D2), f32),
    )(acc2, acc2, b2)


def _sc_edge_pass1(src2, dst2, a1, a1r, h1a, h1b, btile, z80):
    mesh = plsc.VectorSubcoreMesh(
        core_axis_name="c", subcore_axis_name="s", num_cores=NC,
        num_subcores=NS)

    scratch = [
        pltpu.VMEM((CPT1, CH1), i32),
        pltpu.VMEM((CPT1, CH1), i32),
        pltpu.VMEM((16,), f32),
        pltpu.VMEM_SHARED((N, 80), f32),
    ]
    for _ in range(2):
        scratch += [
            pltpu.VMEM((CH1, 16), f32),
            pltpu.VMEM((CH1, 16), f32),
            pltpu.VMEM((CH1, 64), f32),
            pltpu.VMEM((CH1, 80), f32),
            pltpu.SemaphoreType.DMA,
            pltpu.SemaphoreType.DMA,
            pltpu.SemaphoreType.DMA,
        ]

    @functools.partial(
        pl.kernel,
        out_type=jax.ShapeDtypeStruct((NC, N, 80), f32),
        mesh=mesh,
        scratch_types=scratch,
        compiler_params=pltpu.CompilerParams(
            use_tc_tiling_on_sc=False, needs_layout_passes=False),
    )
    def k(src2_hbm, dst2_hbm, a1_hbm, a1r_hbm, h1a_hbm, h1b_hbm, btile_hbm,
          z80_hbm,
          acc_out, sidx_all, didx_all, bvec, sacc,
          arow0, brow0, hrow0, msg0, s00, s01, s02,
          arow1, brow1, hrow1, msg1, s10, s11, s12):
        c = lax.axis_index("c")
        s = lax.axis_index("s")
        arows = (arow0, arow1)
        brows = (brow0, brow1)
        hrows = (hrow0, hrow1)
        msgs = (msg0, msg1)
        sems = ((s00, s01, s02), (s10, s11, s12))
        pltpu.sync_copy(btile_hbm, bvec)
        base = s * CPT1
        pltpu.sync_copy(src2_hbm.at[pl.ds(base, CPT1)], sidx_all)
        pltpu.sync_copy(dst2_hbm.at[pl.ds(base, CPT1)], didx_all)
        _tile_copy(s, lambda r0, nr: pltpu.sync_copy(
            z80_hbm.at[pl.ds(r0, nr)], sacc.at[pl.ds(r0, nr)]))
        plsc.subcore_barrier()
        bv = bvec[...]

        def run_all(cc, h1h_hbm):
            def issue(kk, ab):
                pltpu.async_copy(a1_hbm.at[sidx_all.at[kk]], arows[ab],
                                 sems[ab][0])
                pltpu.async_copy(a1r_hbm.at[didx_all.at[kk]], brows[ab],
                                 sems[ab][1])
                pltpu.async_copy(h1h_hbm.at[sidx_all.at[kk]], hrows[ab],
                                 sems[ab][2])

            def wait_bufs(kk, ab):
                pltpu.make_async_copy(a1_hbm.at[sidx_all.at[kk]], arows[ab],
                                      sems[ab][0]).wait()
                pltpu.make_async_copy(a1r_hbm.at[didx_all.at[kk]], brows[ab],
                                      sems[ab][1]).wait()
                pltpu.make_async_copy(h1h_hbm.at[sidx_all.at[kk]], hrows[ab],
                                      sems[ab][2]).wait()

            def run_chunk(kk, ab):
                wait_bufs(kk, ab)
                arow, brow = arows[ab], brows[ab]
                hrow, msg = hrows[ab], msgs[ab]

                @plsc.parallel_loop(0, CH1, unroll=4)
                def _(e):
                    al = arow[e, :] + brow[e, :]
                    al = jnp.maximum(al, 0.2 * al)
                    ex = jnp.exp(al - bv)
                    msg[e, pl.ds(64, 16)] = ex
                    for hh in range(4):
                        sp = jnp.full((16,), ex[4 * cc + hh], f32)
                        msg[e, pl.ds(hh * 16, 16)] = (
                            hrow[e, pl.ds(hh * 16, 16)] * sp)

                pltpu.sync_copy(msg, sacc.at[didx_all.at[kk]], add=True)

            issue(0, 0)

            def pair_body(p, carry):
                ka = 2 * p
                kb = 2 * p + 1
                issue(kb, 1)
                run_chunk(ka, 0)
                issue(jnp.minimum(ka + 2, CPT1 - 1), 0)
                run_chunk(kb, 1)
                return carry

            lax.fori_loop(0, CPT1 // 2, pair_body, 0)
            wait_bufs(CPT1 - 1, 0)

        @pl.when(c == 0)
        def _():
            run_all(0, h1a_hbm)

        @pl.when(c == 1)
        def _():
            run_all(1, h1b_hbm)

        plsc.subcore_barrier()
        _tile_copy(s, lambda r0, nr: pltpu.sync_copy(
            sacc.at[pl.ds(r0, nr)], acc_out.at[c, pl.ds(r0, nr)]))

    return k(src2, dst2, a1, a1r, h1a, h1b, btile, z80)


def _sc_edge_pass2(src2, dst2, as2, ad2, h2, b2t, z32):
    mesh = plsc.VectorSubcoreMesh(
        core_axis_name="c", subcore_axis_name="s", num_cores=NC,
        num_subcores=NS)

    scratch = [
        pltpu.VMEM((_Q + 1, CH), i32),
        pltpu.VMEM((_Q + 1, CH), i32),
        pltpu.VMEM((N,), f32),
        pltpu.VMEM((N,), f32),
        pltpu.VMEM((16,), f32),
        pltpu.VMEM_SHARED((N, 32), f32),
    ]
    for _ in range(2):
        scratch += [
            pltpu.VMEM((CH, D2), f32),
            pltpu.VMEM((CH, 32), f32),
            pltpu.VMEM((CH,), f32),
            pltpu.SemaphoreType.DMA,
        ]

    @functools.partial(
        pl.kernel,
        out_type=jax.ShapeDtypeStruct((NC, N, 32), f32),
        mesh=mesh,
        scratch_types=scratch,
        compiler_params=pltpu.CompilerParams(
            use_tc_tiling_on_sc=False, needs_layout_passes=False),
    )
    def k(src2_hbm, dst2_hbm, as2_hbm, ad2_hbm, h2_hbm, b2t_hbm, z32_hbm,
          acc_out, sidx_all, didx_all, ast, adt, b2v, sacc2,
          hrow20, msg20, exbuf0, sm0,
          hrow21, msg21, exbuf1, sm1):
        c = lax.axis_index("c")
        s = lax.axis_index("s")
        wid = s * NC + c
        hrows = (hrow20, hrow21)
        msgs = (msg20, msg21)
        exbufs = (exbuf0, exbuf1)
        sems = (sm0, sm1)
        pltpu.sync_copy(b2t_hbm, b2v)
        pltpu.sync_copy(as2_hbm, ast)
        pltpu.sync_copy(ad2_hbm, adt)
        base = wid * _Q
        pltpu.sync_copy(src2_hbm.at[pl.ds(base, _Q)],
                        sidx_all.at[pl.ds(0, _Q)])
        pltpu.sync_copy(dst2_hbm.at[pl.ds(base, _Q)],
                        didx_all.at[pl.ds(0, _Q)])

        @pl.when(wid < _R)
        def _():
            pltpu.sync_copy(src2_hbm.at[NW * _Q + wid], sidx_all.at[_Q])
            pltpu.sync_copy(dst2_hbm.at[NW * _Q + wid], didx_all.at[_Q])

        _tile_copy(s, lambda r0, nr: pltpu.sync_copy(
            z32_hbm.at[pl.ds(r0, nr)], sacc2.at[pl.ds(r0, nr)]))
        plsc.subcore_barrier()
        bv = b2v[...]

        def issue(kk, ab):
            return pltpu.async_copy(h2_hbm.at[sidx_all.at[kk]], hrows[ab],
                                    sems[ab])

        def run_chunk(kk, ab, cp):
            hrow2, msg2, exbuf = hrows[ab], msgs[ab], exbufs[ab]

            @plsc.parallel_loop(0, CH // 16, unroll=2)
            def _(gi):
                sv = sidx_all[kk, pl.ds(gi * 16, 16)]
                dv = didx_all[kk, pl.ds(gi * 16, 16)]
                al = plsc.load_gather(ast, [sv]) + plsc.load_gather(adt, [dv])
                al = jnp.maximum(al, 0.2 * al)
                exbuf[pl.ds(gi * 16, 16)] = jnp.exp(al - bv)

            cp.wait()

            @plsc.parallel_loop(0, CH // 16, unroll=2)
            def _(gi):
                exv = exbuf[pl.ds(gi * 16, 16)]
                for j in range(16):
                    e = gi * 16 + j
                    sp = jnp.full((16,), exv[j], f32)
                    msg2[e, pl.ds(0, 16)] = hrow2[e, :] * sp
                    msg2[e, pl.ds(16, 16)] = sp

            pltpu.sync_copy(msg2, sacc2.at[didx_all.at[kk]], add=True)

        def pair_body(p, carry):
            ka = 2 * p
            kb = 2 * p + 1
            cpa = issue(ka, 0)
            cpb = issue(kb, 1)
            run_chunk(ka, 0, cpa)
            run_chunk(kb, 1, cpb)
            return carry

        lax.fori_loop(0, _Q // 2, pair_body, 0)

        @pl.when(wid < _R)
        def _():
            run_chunk(_Q, 0, issue(_Q, 0))

        plsc.subcore_barrier()
        _tile_copy(s, lambda r0, nr: pltpu.sync_copy(
            sacc2.at[pl.ds(r0, nr)], acc_out.at[c, pl.ds(r0, nr)]))

    return k(src2, dst2, as2, ad2, h2, b2t, z32)


def kernel(x, adj, W1, att_src1, att_dst1, b1, W2, att_src2, att_dst2, b2):
    src = adj[0].astype(i32)
    dst = adj[1].astype(i32)
    src2a = src.reshape(NCHUNK1, CH1)
    dst2a = dst.reshape(NCHUNK1, CH1)
    src2 = src.reshape(NCHUNK, CH)
    dst2 = dst.reshape(NCHUNK, CH)

    eye8 = jnp.eye(HEADS, dtype=f32)
    m_src = (att_src1[0][:, :, None] * eye8[:, None, :]).reshape(D1, HEADS)
    m_dst = (att_dst1[0][:, :, None] * eye8[:, None, :]).reshape(D1, HEADS)
    acat = jnp.concatenate([m_src, m_dst], axis=1)

    h1a, h1b, a1, a1r, bmax1 = _tc1(x, W1, acat)

    bsum = bmax1[0, :8] + bmax1[0, 8:]
    bh = jnp.maximum(bsum, 0.2 * bsum)
    btile = jnp.tile(bh, 2)

    z80 = jnp.zeros((N, 80), f32)
    acc = _sc_edge_pass1(src2a, dst2a, a1, a1r, h1a, h1b, btile, z80)

    e8 = jnp.kron(eye8, jnp.ones((1, HID), f32))
    a2m = jnp.concatenate(
        [att_src2[0, 0][:, None], att_dst2[0, 0][:, None],
         jnp.zeros((D2, 14), f32)], axis=1)
    h2, a2, bmax2 = _tc2(acc, e8, b1.reshape(1, D1), W2, a2m)

    b2sum = bmax2[0, 0] + bmax2[0, 1]
    b2b = jnp.maximum(b2sum, 0.2 * b2sum)
    b2t = jnp.full((16,), b2b, f32)
    as2 = a2[:, 0] + 0.0
    ad2 = a2[:, 1] + 0.0

    z32 = jnp.zeros((N, 32), f32)
    acc2 = _sc_edge_pass2(src2, dst2, as2, ad2, h2, b2t, z32)

    return _tc3(acc2, b2.reshape(1, D2))


# speedup vs baseline: 1.0908x; 1.0412x over previous
import functools

import jax
import jax.numpy as jnp
from jax import lax
from jax.experimental import pallas as pl
from jax.experimental.pallas import tpu as pltpu
from jax.experimental.pallas import tpu_sc as plsc

N = 10000
E = 320000
IN_DIM = 128
HID = 16
HEADS = 8
D1 = HEADS * HID
D2 = 16

NC = 2
NS = 16
NW = NC * NS
CH = 128
NCHUNK = E // CH
CH1 = 100
NCHUNK1 = E // CH1
CPT1 = NCHUNK1 // NS
ROWS_A = 624
ROWS_LAST = N - (NS - 1) * ROWS_A


def _tile_copy(s, copy_fn):
    r0 = s * ROWS_A

    @pl.when(s < NS - 1)
    def _():
        copy_fn(r0, ROWS_A)

    @pl.when(s == NS - 1)
    def _():
        copy_fn(r0, ROWS_LAST)

_Q, _R = divmod(NCHUNK, NW)

f32 = jnp.float32
i32 = jnp.int32


_BLK = 1000
_GRID = N // _BLK


def _tc1_body(x_ref, w1_ref, acat_ref, h1a_ref, h1b_ref, a1_ref, a1r_ref,
              bmax_ref):
    i = pl.program_id(0)
    h = jnp.dot(x_ref[...], w1_ref[...], preferred_element_type=f32)
    h1a_ref[...] = h[:, 0:64]
    h1b_ref[...] = h[:, 64:128]
    a1 = jnp.dot(h, acat_ref[...], preferred_element_type=f32)
    a1_ref[...] = a1
    a1r_ref[...] = jnp.concatenate([a1[:, 8:], a1[:, :8]], axis=1)
    m = jnp.broadcast_to(jnp.max(a1, axis=0, keepdims=True), (8, 16))

    @pl.when(i == 0)
    def _():
        bmax_ref[...] = m

    @pl.when(i != 0)
    def _():
        bmax_ref[...] = jnp.maximum(bmax_ref[...], m)


def _tc1(x, w1, acat):
    return pl.pallas_call(
        _tc1_body,
        grid=(_GRID,),
        in_specs=[
            pl.BlockSpec((_BLK, IN_DIM), lambda i: (i, 0)),
            pl.BlockSpec((IN_DIM, D1), lambda i: (0, 0)),
            pl.BlockSpec((D1, 16), lambda i: (0, 0)),
        ],
        out_specs=[
            pl.BlockSpec((_BLK, 64), lambda i: (i, 0)),
            pl.BlockSpec((_BLK, 64), lambda i: (i, 0)),
            pl.BlockSpec((_BLK, 16), lambda i: (i, 0)),
            pl.BlockSpec((_BLK, 16), lambda i: (i, 0)),
            pl.BlockSpec((8, 16), lambda i: (0, 0)),
        ],
        out_shape=[
            jax.ShapeDtypeStruct((N, 64), f32),
            jax.ShapeDtypeStruct((N, 64), f32),
            jax.ShapeDtypeStruct((N, 16), f32),
            jax.ShapeDtypeStruct((N, 16), f32),
            jax.ShapeDtypeStruct((8, 16), f32),
        ],
    )(x, w1, acat)


def _tc2_body(acc0_ref, acc1_ref, e8_ref, b1_ref, w2_ref, a2m_ref,
              h2_ref, a2_ref, bmax2_ref):
    i = pl.program_id(0)
    a0 = acc0_ref[0]
    a1_ = acc1_ref[0]
    num = jnp.concatenate([a0[:, 0:64], a1_[:, 0:64]], axis=1)
    den8 = a0[:, 64:72]
    den = jnp.dot(den8, e8_ref[...], preferred_element_type=f32)
    out1 = num / (den + 1e-16)
    z = out1 + b1_ref[...]
    z = jnp.where(z > 0, z, jnp.exp(jnp.minimum(z, 0.0)) - 1.0)
    h2 = jnp.dot(z, w2_ref[...], preferred_element_type=f32)
    h2_ref[...] = h2
    a2 = jnp.dot(h2, a2m_ref[...], preferred_element_type=f32)
    a2_ref[...] = a2
    m = jnp.broadcast_to(jnp.max(a2, axis=0, keepdims=True), (8, 16))

    @pl.when(i == 0)
    def _():
        bmax2_ref[...] = m

    @pl.when(i != 0)
    def _():
        bmax2_ref[...] = jnp.maximum(bmax2_ref[...], m)


def _tc2(acc, e8, b1, w2, a2m):
    return pl.pallas_call(
        _tc2_body,
        grid=(_GRID,),
        in_specs=[
            pl.BlockSpec((1, _BLK, 80), lambda i: (0, i, 0)),
            pl.BlockSpec((1, _BLK, 80), lambda i: (1, i, 0)),
            pl.BlockSpec((8, D1), lambda i: (0, 0)),
            pl.BlockSpec((1, D1), lambda i: (0, 0)),
            pl.BlockSpec((D1, D2), lambda i: (0, 0)),
            pl.BlockSpec((D2, 16), lambda i: (0, 0)),
        ],
        out_specs=[
            pl.BlockSpec((_BLK, D2), lambda i: (i, 0)),
            pl.BlockSpec((_BLK, 16), lambda i: (i, 0)),
            pl.BlockSpec((8, 16), lambda i: (0, 0)),
        ],
        out_shape=[
            jax.ShapeDtypeStruct((N, D2), f32),
            jax.ShapeDtypeStruct((N, 16), f32),
            jax.ShapeDtypeStruct((8, 16), f32),
        ],
    )(acc, acc, e8, b1, w2, a2m)


def _tc3_body(acc0_ref, acc1_ref, b2_ref, out_ref):
    a = acc0_ref[0] + acc1_ref[0]
    out_ref[...] = a[:, 0:16] / (a[:, 16:32] + 1e-16) + b2_ref[...]


def _tc3(acc2, b2):
    return pl.pallas_call(
        _tc3_body,
        grid=(_GRID,),
        in_specs=[
            pl.BlockSpec((1, _BLK, 32), lambda i: (0, i, 0)),
            pl.BlockSpec((1, _BLK, 32), lambda i: (1, i, 0)),
            pl.BlockSpec((1, D2), lambda i: (0, 0)),
        ],
        out_specs=pl.BlockSpec((_BLK, D2), lambda i: (i, 0)),
        out_shape=jax.ShapeDtypeStruct((N, D2), f32),
    )(acc2, acc2, b2)


def _sc_edge_pass1(src2, dst2, a1, a1r, h1a, h1b, btile, z80):
    mesh = plsc.VectorSubcoreMesh(
        core_axis_name="c", subcore_axis_name="s", num_cores=NC,
        num_subcores=NS)

    scratch = [
        pltpu.VMEM((CPT1, CH1), i32),
        pltpu.VMEM((CPT1, CH1), i32),
        pltpu.VMEM((16,), f32),
        pltpu.VMEM_SHARED((N, 80), f32),
    ]
    for _ in range(2):
        scratch += [
            pltpu.VMEM((CH1, 16), f32),
            pltpu.VMEM((CH1, 16), f32),
            pltpu.VMEM((CH1, 64), f32),
            pltpu.VMEM((CH1, 80), f32),
            pltpu.SemaphoreType.DMA,
            pltpu.SemaphoreType.DMA,
            pltpu.SemaphoreType.DMA,
        ]

    @functools.partial(
        pl.kernel,
        out_type=jax.ShapeDtypeStruct((NC, N, 80), f32),
        mesh=mesh,
        scratch_types=scratch,
        compiler_params=pltpu.CompilerParams(
            use_tc_tiling_on_sc=False, needs_layout_passes=False),
    )
    def k(src2_hbm, dst2_hbm, a1_hbm, a1r_hbm, h1a_hbm, h1b_hbm, btile_hbm,
          z80_hbm,
          acc_out, sidx_all, didx_all, bvec, sacc,
          arow0, brow0, hrow0, msg0, s00, s01, s02,
          arow1, brow1, hrow1, msg1, s10, s11, s12):
        c = lax.axis_index("c")
        s = lax.axis_index("s")
        arows = (arow0, arow1)
        brows = (brow0, brow1)
        hrows = (hrow0, hrow1)
        msgs = (msg0, msg1)
        sems = ((s00, s01, s02), (s10, s11, s12))
        pltpu.sync_copy(btile_hbm, bvec)
        base = s * CPT1
        pltpu.sync_copy(src2_hbm.at[pl.ds(base, CPT1)], sidx_all)
        pltpu.sync_copy(dst2_hbm.at[pl.ds(base, CPT1)], didx_all)
        _tile_copy(s, lambda r0, nr: pltpu.sync_copy(
            z80_hbm.at[pl.ds(r0, nr)], sacc.at[pl.ds(r0, nr)]))
        plsc.subcore_barrier()
        bv = bvec[...]

        def run_all(cc, h1h_hbm):
            def issue(kk, ab):
                pltpu.async_copy(a1_hbm.at[sidx_all.at[kk]], arows[ab],
                                 sems[ab][0])
                pltpu.async_copy(a1r_hbm.at[didx_all.at[kk]], brows[ab],
                                 sems[ab][1])
                pltpu.async_copy(h1h_hbm.at[sidx_all.at[kk]], hrows[ab],
                                 sems[ab][2])

            def wait_bufs(kk, ab):
                pltpu.make_async_copy(a1_hbm.at[sidx_all.at[kk]], arows[ab],
                                      sems[ab][0]).wait()
                pltpu.make_async_copy(a1r_hbm.at[didx_all.at[kk]], brows[ab],
                                      sems[ab][1]).wait()
                pltpu.make_async_copy(h1h_hbm.at[sidx_all.at[kk]], hrows[ab],
                                      sems[ab][2]).wait()

            def run_chunk(kk, ab):
                wait_bufs(kk, ab)
                arow, brow = arows[ab], brows[ab]
                hrow, msg = hrows[ab], msgs[ab]

                @plsc.parallel_loop(0, CH1, unroll=4)
                def _(e):
                    al = arow[e, :] + brow[e, :]
                    al = jnp.maximum(al, 0.2 * al)
                    ex = jnp.exp(al - bv)
                    msg[e, pl.ds(64, 16)] = ex
                    for hh in range(4):
                        sp = jnp.full((16,), ex[4 * cc + hh], f32)
                        msg[e, pl.ds(hh * 16, 16)] = (
                            hrow[e, pl.ds(hh * 16, 16)] * sp)

                pltpu.sync_copy(msg, sacc.at[didx_all.at[kk]], add=True)

            issue(0, 0)

            def pair_body(p, carry):
                ka = 2 * p
                kb = 2 * p + 1
                issue(kb, 1)
                run_chunk(ka, 0)
                issue(jnp.minimum(ka + 2, CPT1 - 1), 0)
                run_chunk(kb, 1)
                return carry

            lax.fori_loop(0, CPT1 // 2, pair_body, 0)
            wait_bufs(CPT1 - 1, 0)

        @pl.when(c == 0)
        def _():
            run_all(0, h1a_hbm)

        @pl.when(c == 1)
        def _():
            run_all(1, h1b_hbm)

        plsc.subcore_barrier()
        _tile_copy(s, lambda r0, nr: pltpu.sync_copy(
            sacc.at[pl.ds(r0, nr)], acc_out.at[c, pl.ds(r0, nr)]))

    return k(src2, dst2, a1, a1r, h1a, h1b, btile, z80)


def _sc_edge_pass2(src2, dst2, as2, ad2, h2, b2t, z32):
    mesh = plsc.VectorSubcoreMesh(
        core_axis_name="c", subcore_axis_name="s", num_cores=NC,
        num_subcores=NS)

    scratch = [
        pltpu.VMEM((_Q + 1, CH), i32),
        pltpu.VMEM((_Q + 1, CH), i32),
        pltpu.VMEM((N,), f32),
        pltpu.VMEM((N,), f32),
        pltpu.VMEM((16,), f32),
        pltpu.VMEM_SHARED((N, 32), f32),
    ]
    for _ in range(2):
        scratch += [
            pltpu.VMEM((CH, D2), f32),
            pltpu.VMEM((CH, 32), f32),
            pltpu.VMEM((CH,), f32),
            pltpu.SemaphoreType.DMA,
        ]

    @functools.partial(
        pl.kernel,
        out_type=jax.ShapeDtypeStruct((NC, N, 32), f32),
        mesh=mesh,
        scratch_types=scratch,
        compiler_params=pltpu.CompilerParams(
            use_tc_tiling_on_sc=False, needs_layout_passes=False),
    )
    def k(src2_hbm, dst2_hbm, as2_hbm, ad2_hbm, h2_hbm, b2t_hbm, z32_hbm,
          acc_out, sidx_all, didx_all, ast, adt, b2v, sacc2,
          hrow20, msg20, exbuf0, sm0,
          hrow21, msg21, exbuf1, sm1):
        c = lax.axis_index("c")
        s = lax.axis_index("s")
        wid = s * NC + c
        hrows = (hrow20, hrow21)
        msgs = (msg20, msg21)
        exbufs = (exbuf0, exbuf1)
        sems = (sm0, sm1)
        pltpu.sync_copy(b2t_hbm, b2v)
        pltpu.sync_copy(as2_hbm, ast)
        pltpu.sync_copy(ad2_hbm, adt)
        base = wid * _Q
        pltpu.sync_copy(src2_hbm.at[pl.ds(base, _Q)],
                        sidx_all.at[pl.ds(0, _Q)])
        pltpu.sync_copy(dst2_hbm.at[pl.ds(base, _Q)],
                        didx_all.at[pl.ds(0, _Q)])

        @pl.when(wid < _R)
        def _():
            pltpu.sync_copy(src2_hbm.at[NW * _Q + wid], sidx_all.at[_Q])
            pltpu.sync_copy(dst2_hbm.at[NW * _Q + wid], didx_all.at[_Q])

        _tile_copy(s, lambda r0, nr: pltpu.sync_copy(
            z32_hbm.at[pl.ds(r0, nr)], sacc2.at[pl.ds(r0, nr)]))
        plsc.subcore_barrier()
        bv = b2v[...]

        def issue(kk, ab):
            pltpu.async_copy(h2_hbm.at[sidx_all.at[kk]], hrows[ab], sems[ab])

        def run_chunk(kk, ab):
            hrow2, msg2, exbuf = hrows[ab], msgs[ab], exbufs[ab]

            @plsc.parallel_loop(0, CH // 16, unroll=2)
            def _(gi):
                sv = sidx_all[kk, pl.ds(gi * 16, 16)]
                dv = didx_all[kk, pl.ds(gi * 16, 16)]
                al = plsc.load_gather(ast, [sv]) + plsc.load_gather(adt, [dv])
                al = jnp.maximum(al, 0.2 * al)
                exbuf[pl.ds(gi * 16, 16)] = jnp.exp(al - bv)

            pltpu.make_async_copy(h2_hbm.at[sidx_all.at[kk]], hrows[ab],
                                  sems[ab]).wait()

            @plsc.parallel_loop(0, CH // 16, unroll=2)
            def _(gi):
                exv = exbuf[pl.ds(gi * 16, 16)]
                for j in range(16):
                    e = gi * 16 + j
                    sp = jnp.full((16,), exv[j], f32)
                    msg2[e, pl.ds(0, 16)] = hrow2[e, :] * sp
                    msg2[e, pl.ds(16, 16)] = sp

            pltpu.sync_copy(msg2, sacc2.at[didx_all.at[kk]], add=True)

        issue(0, 0)

        def pair_body(p, carry):
            ka = 2 * p
            kb = 2 * p + 1
            issue(kb, 1)
            run_chunk(ka, 0)
            issue(jnp.minimum(ka + 2, _Q - 1), 0)
            run_chunk(kb, 1)
            return carry

        lax.fori_loop(0, _Q // 2, pair_body, 0)
        pltpu.make_async_copy(h2_hbm.at[sidx_all.at[_Q - 1]], hrows[0],
                              sems[0]).wait()

        @pl.when(wid < _R)
        def _():
            issue(_Q, 0)
            run_chunk(_Q, 0)

        plsc.subcore_barrier()
        _tile_copy(s, lambda r0, nr: pltpu.sync_copy(
            sacc2.at[pl.ds(r0, nr)], acc_out.at[c, pl.ds(r0, nr)]))

    return k(src2, dst2, as2, ad2, h2, b2t, z32)


def kernel(x, adj, W1, att_src1, att_dst1, b1, W2, att_src2, att_dst2, b2):
    src = adj[0].astype(i32)
    dst = adj[1].astype(i32)
    src2a = src.reshape(NCHUNK1, CH1)
    dst2a = dst.reshape(NCHUNK1, CH1)
    src2 = src.reshape(NCHUNK, CH)
    dst2 = dst.reshape(NCHUNK, CH)

    eye8 = jnp.eye(HEADS, dtype=f32)
    m_src = (att_src1[0][:, :, None] * eye8[:, None, :]).reshape(D1, HEADS)
    m_dst = (att_dst1[0][:, :, None] * eye8[:, None, :]).reshape(D1, HEADS)
    acat = jnp.concatenate([m_src, m_dst], axis=1)

    h1a, h1b, a1, a1r, bmax1 = _tc1(x, W1, acat)

    bsum = bmax1[0, :8] + bmax1[0, 8:]
    bh = jnp.maximum(bsum, 0.2 * bsum)
    btile = jnp.tile(bh, 2)

    z80 = jnp.zeros((N, 80), f32)
    acc = _sc_edge_pass1(src2a, dst2a, a1, a1r, h1a, h1b, btile, z80)

    e8 = jnp.kron(eye8, jnp.ones((1, HID), f32))
    a2m = jnp.concatenate(
        [att_src2[0, 0][:, None], att_dst2[0, 0][:, None],
         jnp.zeros((D2, 14), f32)], axis=1)
    h2, a2, bmax2 = _tc2(acc, e8, b1.reshape(1, D1), W2, a2m)

    b2sum = bmax2[0, 0] + bmax2[0, 1]
    b2b = jnp.maximum(b2sum, 0.2 * b2sum)
    b2t = jnp.full((16,), b2b, f32)
    as2 = a2[:, 0] + 0.0
    ad2 = a2[:, 1] + 0.0

    z32 = jnp.zeros((N, 32), f32)
    acc2 = _sc_edge_pass2(src2, dst2, as2, ad2, h2, b2t, z32)

    return _tc3(acc2, b2.reshape(1, D2))


# speedup vs baseline: 1.0927x; 1.0017x over previous
import functools

import jax
import jax.numpy as jnp
from jax import lax
from jax.experimental import pallas as pl
from jax.experimental.pallas import tpu as pltpu
from jax.experimental.pallas import tpu_sc as plsc

N = 10000
E = 320000
IN_DIM = 128
HID = 16
HEADS = 8
D1 = HEADS * HID
D2 = 16

NC = 2
NS = 16
NW = NC * NS
CH = 128
NCHUNK = E // CH
CH1 = 100
NCHUNK1 = E // CH1
CPT1 = NCHUNK1 // NS
ROWS_A = 624
ROWS_LAST = N - (NS - 1) * ROWS_A


def _tile_copy(s, copy_fn):
    r0 = s * ROWS_A

    @pl.when(s < NS - 1)
    def _():
        copy_fn(r0, ROWS_A)

    @pl.when(s == NS - 1)
    def _():
        copy_fn(r0, ROWS_LAST)

_Q, _R = divmod(NCHUNK, NW)

f32 = jnp.float32
i32 = jnp.int32


_BLK = 1000
_GRID = N // _BLK


def _tc1_body(x_ref, w1_ref, acat_ref, h1a_ref, h1b_ref, a1_ref, a1r_ref,
              bmax_ref):
    i = pl.program_id(0)
    h = jnp.dot(x_ref[...], w1_ref[...], preferred_element_type=f32)
    h1a_ref[...] = h[:, 0:64]
    h1b_ref[...] = h[:, 64:128]
    a1 = jnp.dot(h, acat_ref[...], preferred_element_type=f32)
    a1_ref[...] = a1
    a1r_ref[...] = jnp.concatenate([a1[:, 8:], a1[:, :8]], axis=1)
    m = jnp.broadcast_to(jnp.max(a1, axis=0, keepdims=True), (8, 16))

    @pl.when(i == 0)
    def _():
        bmax_ref[...] = m

    @pl.when(i != 0)
    def _():
        bmax_ref[...] = jnp.maximum(bmax_ref[...], m)


def _tc1(x, w1, acat):
    return pl.pallas_call(
        _tc1_body,
        grid=(_GRID,),
        in_specs=[
            pl.BlockSpec((_BLK, IN_DIM), lambda i: (i, 0)),
            pl.BlockSpec((IN_DIM, D1), lambda i: (0, 0)),
            pl.BlockSpec((D1, 16), lambda i: (0, 0)),
        ],
        out_specs=[
            pl.BlockSpec((_BLK, 64), lambda i: (i, 0)),
            pl.BlockSpec((_BLK, 64), lambda i: (i, 0)),
            pl.BlockSpec((_BLK, 16), lambda i: (i, 0)),
            pl.BlockSpec((_BLK, 16), lambda i: (i, 0)),
            pl.BlockSpec((8, 16), lambda i: (0, 0)),
        ],
        out_shape=[
            jax.ShapeDtypeStruct((N, 64), f32),
            jax.ShapeDtypeStruct((N, 64), f32),
            jax.ShapeDtypeStruct((N, 16), f32),
            jax.ShapeDtypeStruct((N, 16), f32),
            jax.ShapeDtypeStruct((8, 16), f32),
        ],
    )(x, w1, acat)


def _tc2_body(acc0_ref, acc1_ref, e8_ref, b1_ref, w2_ref, a2m_ref,
              h2_ref, a2_ref, bmax2_ref):
    i = pl.program_id(0)
    a0 = acc0_ref[0]
    a1_ = acc1_ref[0]
    num = jnp.concatenate([a0[:, 0:64], a1_[:, 0:64]], axis=1)
    den8 = a0[:, 64:72]
    den = jnp.dot(den8, e8_ref[...], preferred_element_type=f32)
    out1 = num / (den + 1e-16)
    z = out1 + b1_ref[...]
    z = jnp.where(z > 0, z, jnp.exp(jnp.minimum(z, 0.0)) - 1.0)
    h2 = jnp.dot(z, w2_ref[...], preferred_element_type=f32)
    h2_ref[...] = h2
    a2 = jnp.dot(h2, a2m_ref[...], preferred_element_type=f32)
    a2_ref[...] = a2
    m = jnp.broadcast_to(jnp.max(a2, axis=0, keepdims=True), (8, 16))

    @pl.when(i == 0)
    def _():
        bmax2_ref[...] = m

    @pl.when(i != 0)
    def _():
        bmax2_ref[...] = jnp.maximum(bmax2_ref[...], m)


def _tc2(acc, e8, b1, w2, a2m):
    return pl.pallas_call(
        _tc2_body,
        grid=(_GRID,),
        in_specs=[
            pl.BlockSpec((1, _BLK, 80), lambda i: (0, i, 0)),
            pl.BlockSpec((1, _BLK, 80), lambda i: (1, i, 0)),
            pl.BlockSpec((8, D1), lambda i: (0, 0)),
            pl.BlockSpec((1, D1), lambda i: (0, 0)),
            pl.BlockSpec((D1, D2), lambda i: (0, 0)),
            pl.BlockSpec((D2, 16), lambda i: (0, 0)),
        ],
        out_specs=[
            pl.BlockSpec((_BLK, D2), lambda i: (i, 0)),
            pl.BlockSpec((_BLK, 16), lambda i: (i, 0)),
            pl.BlockSpec((8, 16), lambda i: (0, 0)),
        ],
        out_shape=[
            jax.ShapeDtypeStruct((N, D2), f32),
            jax.ShapeDtypeStruct((N, 16), f32),
            jax.ShapeDtypeStruct((8, 16), f32),
        ],
    )(acc, acc, e8, b1, w2, a2m)


def _tc3_body(acc0_ref, acc1_ref, b2_ref, out_ref):
    a = acc0_ref[0] + acc1_ref[0]
    out_ref[...] = a[:, 0:16] / (a[:, 16:32] + 1e-16) + b2_ref[...]


def _tc3(acc2, b2):
    return pl.pallas_call(
        _tc3_body,
        grid=(_GRID,),
        in_specs=[
            pl.BlockSpec((1, _BLK, 32), lambda i: (0, i, 0)),
            pl.BlockSpec((1, _BLK, 32), lambda i: (1, i, 0)),
            pl.BlockSpec((1, D2), lambda i: (0, 0)),
        ],
        out_specs=pl.BlockSpec((_BLK, D2), lambda i: (i, 0)),
        out_shape=jax.ShapeDtypeStruct((N, D2), f32),
    )(acc2, acc2, b2)


def _sc_edge_pass1(src2, dst2, a1, a1r, h1a, h1b, btile, z80):
    mesh = plsc.VectorSubcoreMesh(
        core_axis_name="c", subcore_axis_name="s", num_cores=NC,
        num_subcores=NS)

    scratch = [
        pltpu.VMEM((CPT1, CH1), i32),
        pltpu.VMEM((CPT1, CH1), i32),
        pltpu.VMEM((16,), f32),
        pltpu.VMEM_SHARED((N, 80), f32),
    ]
    for _ in range(2):
        scratch += [
            pltpu.VMEM((CH1, 16), f32),
            pltpu.VMEM((CH1, 16), f32),
            pltpu.VMEM((CH1, 64), f32),
            pltpu.VMEM((CH1, 80), f32),
            pltpu.SemaphoreType.DMA,
            pltpu.SemaphoreType.DMA,
            pltpu.SemaphoreType.DMA,
        ]

    @functools.partial(
        pl.kernel,
        out_type=jax.ShapeDtypeStruct((NC, N, 80), f32),
        mesh=mesh,
        scratch_types=scratch,
        compiler_params=pltpu.CompilerParams(
            use_tc_tiling_on_sc=False, needs_layout_passes=False),
    )
    def k(src2_hbm, dst2_hbm, a1_hbm, a1r_hbm, h1a_hbm, h1b_hbm, btile_hbm,
          z80_hbm,
          acc_out, sidx_all, didx_all, bvec, sacc,
          arow0, brow0, hrow0, msg0, s00, s01, s02,
          arow1, brow1, hrow1, msg1, s10, s11, s12):
        c = lax.axis_index("c")
        s = lax.axis_index("s")
        arows = (arow0, arow1)
        brows = (brow0, brow1)
        hrows = (hrow0, hrow1)
        msgs = (msg0, msg1)
        sems = ((s00, s01, s02), (s10, s11, s12))
        pltpu.sync_copy(btile_hbm, bvec)
        base = s * CPT1
        pltpu.sync_copy(src2_hbm.at[pl.ds(base, CPT1)], sidx_all)
        pltpu.sync_copy(dst2_hbm.at[pl.ds(base, CPT1)], didx_all)
        _tile_copy(s, lambda r0, nr: pltpu.sync_copy(
            z80_hbm.at[pl.ds(r0, nr)], sacc.at[pl.ds(r0, nr)]))
        plsc.subcore_barrier()
        bv = bvec[...]

        def run_all(cc, h1h_hbm):
            def issue(kk, ab):
                pltpu.async_copy(a1_hbm.at[sidx_all.at[kk]], arows[ab],
                                 sems[ab][0])
                pltpu.async_copy(a1r_hbm.at[didx_all.at[kk]], brows[ab],
                                 sems[ab][1])
                pltpu.async_copy(h1h_hbm.at[sidx_all.at[kk]], hrows[ab],
                                 sems[ab][2])

            def wait_bufs(kk, ab):
                pltpu.make_async_copy(a1_hbm.at[sidx_all.at[kk]], arows[ab],
                                      sems[ab][0]).wait()
                pltpu.make_async_copy(a1r_hbm.at[didx_all.at[kk]], brows[ab],
                                      sems[ab][1]).wait()
                pltpu.make_async_copy(h1h_hbm.at[sidx_all.at[kk]], hrows[ab],
                                      sems[ab][2]).wait()

            def run_chunk(kk, ab):
                wait_bufs(kk, ab)
                arow, brow = arows[ab], brows[ab]
                hrow, msg = hrows[ab], msgs[ab]

                @plsc.parallel_loop(0, CH1, unroll=2)
                def _(e):
                    al = arow[e, :] + brow[e, :]
                    al = jnp.maximum(al, 0.2 * al)
                    ex = jnp.exp(al - bv)
                    msg[e, pl.ds(64, 16)] = ex
                    for hh in range(4):
                        sp = jnp.full((16,), ex[4 * cc + hh], f32)
                        msg[e, pl.ds(hh * 16, 16)] = (
                            hrow[e, pl.ds(hh * 16, 16)] * sp)

                pltpu.sync_copy(msg, sacc.at[didx_all.at[kk]], add=True)

            issue(0, 0)

            def pair_body(p, carry):
                ka = 2 * p
                kb = 2 * p + 1
                issue(kb, 1)
                run_chunk(ka, 0)
                issue(jnp.minimum(ka + 2, CPT1 - 1), 0)
                run_chunk(kb, 1)
                return carry

            lax.fori_loop(0, CPT1 // 2, pair_body, 0)
            wait_bufs(CPT1 - 1, 0)

        @pl.when(c == 0)
        def _():
            run_all(0, h1a_hbm)

        @pl.when(c == 1)
        def _():
            run_all(1, h1b_hbm)

        plsc.subcore_barrier()
        _tile_copy(s, lambda r0, nr: pltpu.sync_copy(
            sacc.at[pl.ds(r0, nr)], acc_out.at[c, pl.ds(r0, nr)]))

    return k(src2, dst2, a1, a1r, h1a, h1b, btile, z80)


def _sc_edge_pass2(src2, dst2, as2, ad2, h2, b2t, z32):
    mesh = plsc.VectorSubcoreMesh(
        core_axis_name="c", subcore_axis_name="s", num_cores=NC,
        num_subcores=NS)

    scratch = [
        pltpu.VMEM((_Q + 1, CH), i32),
        pltpu.VMEM((_Q + 1, CH), i32),
        pltpu.VMEM((N,), f32),
        pltpu.VMEM((N,), f32),
        pltpu.VMEM((16,), f32),
        pltpu.VMEM_SHARED((N, 32), f32),
    ]
    for _ in range(2):
        scratch += [
            pltpu.VMEM((CH, D2), f32),
            pltpu.VMEM((CH, 32), f32),
            pltpu.VMEM((CH,), f32),
            pltpu.SemaphoreType.DMA,
        ]

    @functools.partial(
        pl.kernel,
        out_type=jax.ShapeDtypeStruct((NC, N, 32), f32),
        mesh=mesh,
        scratch_types=scratch,
        compiler_params=pltpu.CompilerParams(
            use_tc_tiling_on_sc=False, needs_layout_passes=False),
    )
    def k(src2_hbm, dst2_hbm, as2_hbm, ad2_hbm, h2_hbm, b2t_hbm, z32_hbm,
          acc_out, sidx_all, didx_all, ast, adt, b2v, sacc2,
          hrow20, msg20, exbuf0, sm0,
          hrow21, msg21, exbuf1, sm1):
        c = lax.axis_index("c")
        s = lax.axis_index("s")
        wid = s * NC + c
        hrows = (hrow20, hrow21)
        msgs = (msg20, msg21)
        exbufs = (exbuf0, exbuf1)
        sems = (sm0, sm1)
        pltpu.sync_copy(b2t_hbm, b2v)
        pltpu.sync_copy(as2_hbm, ast)
        pltpu.sync_copy(ad2_hbm, adt)
        base = wid * _Q
        pltpu.sync_copy(src2_hbm.at[pl.ds(base, _Q)],
                        sidx_all.at[pl.ds(0, _Q)])
        pltpu.sync_copy(dst2_hbm.at[pl.ds(base, _Q)],
                        didx_all.at[pl.ds(0, _Q)])

        @pl.when(wid < _R)
        def _():
            pltpu.sync_copy(src2_hbm.at[NW * _Q + wid], sidx_all.at[_Q])
            pltpu.sync_copy(dst2_hbm.at[NW * _Q + wid], didx_all.at[_Q])

        _tile_copy(s, lambda r0, nr: pltpu.sync_copy(
            z32_hbm.at[pl.ds(r0, nr)], sacc2.at[pl.ds(r0, nr)]))
        plsc.subcore_barrier()
        bv = b2v[...]

        def issue(kk, ab):
            pltpu.async_copy(h2_hbm.at[sidx_all.at[kk]], hrows[ab], sems[ab])

        def run_chunk(kk, ab):
            hrow2, msg2, exbuf = hrows[ab], msgs[ab], exbufs[ab]

            @plsc.parallel_loop(0, CH // 16, unroll=2)
            def _(gi):
                sv = sidx_all[kk, pl.ds(gi * 16, 16)]
                dv = didx_all[kk, pl.ds(gi * 16, 16)]
                al = plsc.load_gather(ast, [sv]) + plsc.load_gather(adt, [dv])
                al = jnp.maximum(al, 0.2 * al)
                exbuf[pl.ds(gi * 16, 16)] = jnp.exp(al - bv)

            pltpu.make_async_copy(h2_hbm.at[sidx_all.at[kk]], hrows[ab],
                                  sems[ab]).wait()

            @plsc.parallel_loop(0, CH // 16, unroll=2)
            def _(gi):
                exv = exbuf[pl.ds(gi * 16, 16)]
                for j in range(16):
                    e = gi * 16 + j
                    sp = jnp.full((16,), exv[j], f32)
                    msg2[e, pl.ds(0, 16)] = hrow2[e, :] * sp
                    msg2[e, pl.ds(16, 16)] = sp

            pltpu.sync_copy(msg2, sacc2.at[didx_all.at[kk]], add=True)

        issue(0, 0)

        def pair_body(p, carry):
            ka = 2 * p
            kb = 2 * p + 1
            issue(kb, 1)
            run_chunk(ka, 0)
            issue(jnp.minimum(ka + 2, _Q - 1), 0)
            run_chunk(kb, 1)
            return carry

        lax.fori_loop(0, _Q // 2, pair_body, 0)
        pltpu.make_async_copy(h2_hbm.at[sidx_all.at[_Q - 1]], hrows[0],
                              sems[0]).wait()

        @pl.when(wid < _R)
        def _():
            issue(_Q, 0)
            run_chunk(_Q, 0)

        plsc.subcore_barrier()
        _tile_copy(s, lambda r0, nr: pltpu.sync_copy(
            sacc2.at[pl.ds(r0, nr)], acc_out.at[c, pl.ds(r0, nr)]))

    return k(src2, dst2, as2, ad2, h2, b2t, z32)


def kernel(x, adj, W1, att_src1, att_dst1, b1, W2, att_src2, att_dst2, b2):
    src = adj[0].astype(i32)
    dst = adj[1].astype(i32)
    src2a = src.reshape(NCHUNK1, CH1)
    dst2a = dst.reshape(NCHUNK1, CH1)
    src2 = src.reshape(NCHUNK, CH)
    dst2 = dst.reshape(NCHUNK, CH)

    eye8 = jnp.eye(HEADS, dtype=f32)
    m_src = (att_src1[0][:, :, None] * eye8[:, None, :]).reshape(D1, HEADS)
    m_dst = (att_dst1[0][:, :, None] * eye8[:, None, :]).reshape(D1, HEADS)
    acat = jnp.concatenate([m_src, m_dst], axis=1)

    h1a, h1b, a1, a1r, bmax1 = _tc1(x, W1, acat)

    bsum = bmax1[0, :8] + bmax1[0, 8:]
    bh = jnp.maximum(bsum, 0.2 * bsum)
    btile = jnp.tile(bh, 2)

    z80 = jnp.zeros((N, 80), f32)
    acc = _sc_edge_pass1(src2a, dst2a, a1, a1r, h1a, h1b, btile, z80)

    e8 = jnp.kron(eye8, jnp.ones((1, HID), f32))
    a2m = jnp.concatenate(
        [att_src2[0, 0][:, None], att_dst2[0, 0][:, None],
         jnp.zeros((D2, 14), f32)], axis=1)
    h2, a2, bmax2 = _tc2(acc, e8, b1.reshape(1, D1), W2, a2m)

    b2sum = bmax2[0, 0] + bmax2[0, 1]
    b2b = jnp.maximum(b2sum, 0.2 * b2sum)
    b2t = jnp.full((16,), b2b, f32)
    as2 = a2[:, 0] + 0.0
    ad2 = a2[:, 1] + 0.0

    z32 = jnp.zeros((N, 32), f32)
    acc2 = _sc_edge_pass2(src2, dst2, as2, ad2, h2, b2t, z32)

    return _tc3(acc2, b2.reshape(1, D2))
